# Initial kernel scaffold; baseline (speedup 1.0000x reference)
import jax
import jax.numpy as jnp
from jax.experimental import pallas as pl


def kernel(adj_indices, adj_values, user_emb, item_emb):
    raise NotImplementedError("write your pallas kernel here")


# speedup vs baseline: 2.6858x; 2.6858x over previous
import functools

import jax
import jax.numpy as jnp
from jax import lax
from jax.experimental import pallas as pl
from jax.experimental.pallas import tpu as pltpu
from jax.experimental.pallas import tpu_sc as plsc

NU = 25000
NI = 25000
NN = NU + NI
D = 64
E = 800000
HALF = NN // 2

CH = 128
NCH = 8
SG = CH * NCH
OUTER = 50
PT = SG * OUTER
EP = PT * 16
EROWS = EP // CH

ACC_ROWS = 25088
ZR = ACC_ROWS // 16
TAIL = HALF - 15 * ZR
ZB = 224


def _layer_body(x_hbm, src_hbm, dst_hbm, val_hbm, out_hbm,
                src_st, dst_st, val_st, loc_st, rows_v, zrow_v, acc, sem):
    c = lax.axis_index("c")
    s = lax.axis_index("s")
    lo = c * HALF

    def zrow_body(r, carry):
        for j in range(D // 16):
            zrow_v[r, pl.ds(j * 16, 16)] = jnp.zeros((16,), jnp.float32)
        return carry
    lax.fori_loop(0, ZB, zrow_body, 0)
    zbase = s * ZR
    for k in range(ZR // ZB):
        pltpu.sync_copy(zrow_v, acc.at[pl.ds(zbase + k * ZB, ZB)])
    plsc.subcore_barrier()

    rbase = s * (PT // CH)

    def outer_body(o, carry):
        rb = rbase + o * NCH
        pltpu.sync_copy(src_hbm.at[pl.ds(rb, NCH)], src_st)
        pltpu.sync_copy(dst_hbm.at[pl.ds(rb, NCH)], dst_st)
        pltpu.sync_copy(val_hbm.at[pl.ds(rb, NCH)], val_st)

        for k in range(NCH):
            for j in range(CH // 16):
                d = dst_st[k, pl.ds(j * 16, 16)]
                in_r = (d >= lo) & (d < lo + HALF)
                dump = HALF + (d & 63)
                loc_st[k, pl.ds(j * 16, 16)] = jnp.where(in_r, d - lo, dump)

        for k in range(NCH):
            pltpu.async_copy(x_hbm.at[src_st.at[k]], rows_v, sem).wait()

            def scale_group(g, carry2):
                vvals = val_st[k, pl.ds(g * 16, 16)]
                base = g * 16
                for i in range(16):
                    vv = jnp.broadcast_to(vvals[i], (16,))
                    for j in range(D // 16):
                        sl = pl.ds(j * 16, 16)
                        rows_v[base + i, sl] = rows_v[base + i, sl] * vv
                return carry2
            lax.fori_loop(0, CH // 16, scale_group, 0)

            pltpu.sync_copy(rows_v, acc.at[loc_st.at[k]], add=True)
        return carry

    lax.fori_loop(0, OUTER, outer_body, 0)
    plsc.subcore_barrier()

    @pl.when(s < 15)
    def _copy_full():
        pltpu.sync_copy(acc.at[pl.ds(s * ZR, ZR)],
                        out_hbm.at[pl.ds(lo + s * ZR, ZR)])

    @pl.when(s == 15)
    def _copy_tail():
        pltpu.sync_copy(acc.at[pl.ds(15 * ZR, TAIL)],
                        out_hbm.at[pl.ds(lo + 15 * ZR, TAIL)])


_layer = functools.partial(
    pl.kernel,
    out_type=jax.ShapeDtypeStruct((NN, D), jnp.float32),
    mesh=plsc.VectorSubcoreMesh(core_axis_name="c", subcore_axis_name="s",
                                num_cores=2, num_subcores=16),
    compiler_params=pltpu.CompilerParams(use_tc_tiling_on_sc=False),
    scratch_types=[
        pltpu.VMEM((NCH, CH), jnp.int32),
        pltpu.VMEM((NCH, CH), jnp.int32),
        pltpu.VMEM((NCH, CH), jnp.float32),
        pltpu.VMEM((NCH, CH), jnp.int32),
        pltpu.VMEM((CH, D), jnp.float32),
        pltpu.VMEM((ZB, D), jnp.float32),
        pltpu.VMEM_SHARED((ACC_ROWS, D), jnp.float32),
        pltpu.SemaphoreType.DMA,
    ],
)(_layer_body)


def _mean_body(a_ref, b_ref, c_ref, d_ref, o_ref):
    o_ref[...] = (a_ref[...] + b_ref[...] + c_ref[...] + d_ref[...]) * 0.25


def _mean4(a, b, c, d):
    blk = (400, D)
    spec = pl.BlockSpec(blk, lambda i: (i, 0))
    return pl.pallas_call(
        _mean_body,
        grid=(NN // blk[0],),
        in_specs=[spec] * 4,
        out_specs=spec,
        out_shape=jax.ShapeDtypeStruct((NN, D), jnp.float32),
    )(a, b, c, d)


def kernel(adj_indices, adj_values, user_emb, item_emb):
    x0 = jnp.concatenate([user_emb, item_emb], axis=0)
    dst = adj_indices[0].astype(jnp.int32)
    src = adj_indices[1].astype(jnp.int32)
    pad = EP - E
    src2 = jnp.concatenate([src, jnp.zeros((pad,), jnp.int32)]).reshape(EROWS, CH)
    dst2 = jnp.concatenate([dst, jnp.full((pad,), NN, jnp.int32)]).reshape(EROWS, CH)
    val2 = jnp.concatenate([adj_values.astype(jnp.float32),
                            jnp.zeros((pad,), jnp.float32)]).reshape(EROWS, CH)

    x1 = _layer(x0, src2, dst2, val2)
    x2 = _layer(x1, src2, dst2, val2)
    x3 = _layer(x2, src2, dst2, val2)
    out = _mean4(x0, x1, x2, x3)
    return out[:NU], out[NU:]


# speedup vs baseline: 3.6323x; 1.3524x over previous
import functools

import jax
import jax.numpy as jnp
from jax import lax
from jax.experimental import pallas as pl
from jax.experimental.pallas import tpu as pltpu
from jax.experimental.pallas import tpu_sc as plsc

NU = 25000
NI = 25000
NN = NU + NI
D = 64
E = 800000
HALF = NN // 2

CH = 128
NCH = 4
SG = CH * NCH
NST = 100
NCHT = NCH * NST
PT = SG * NST
EP = PT * 16
EROWS = EP // CH

NB = 3
LOOK = 2

ACC_ROWS = 25088
ZR = ACC_ROWS // 16
TAIL = HALF - 15 * ZR


def _layer_body(x_hbm, src_hbm, dst_hbm, val_hbm, out_hbm,
                src_st, dst_st, val_st, rows_v, acc, semg, sems, semt):
    c = lax.axis_index("c")
    s = lax.axis_index("s")

    def zrow_body(r, carry):
        for j in range(D // 16):
            rows_v[0, r, pl.ds(j * 16, 16)] = jnp.zeros((16,), jnp.float32)
        return carry
    lax.fori_loop(0, CH, zrow_body, 0)
    zbase = s * ZR
    for k in range(ZR // CH):
        pltpu.sync_copy(rows_v.at[0], acc.at[pl.ds(zbase + k * CH, CH)])
    pltpu.sync_copy(rows_v.at[0, pl.ds(0, ZR % CH)],
                    acc.at[pl.ds(zbase + (ZR // CH) * CH, ZR % CH)])
    plsc.subcore_barrier()

    rbase = s * (PT // CH)

    def stage_fire(q, p):
        rb = rbase + q * NCH
        pltpu.async_copy(src_hbm.at[pl.ds(rb, NCH)], src_st.at[p], semt)
        pltpu.async_copy(dst_hbm.at[pl.ds(rb, NCH)], dst_st.at[p], semt)
        pltpu.async_copy(val_hbm.at[pl.ds(rb, NCH)], val_st.at[p], semt)

    def stage_wait(p):
        rb = rbase
        pltpu.make_async_copy(src_hbm.at[pl.ds(rb, NCH)], src_st.at[p], semt).wait()
        pltpu.make_async_copy(dst_hbm.at[pl.ds(rb, NCH)], dst_st.at[p], semt).wait()
        pltpu.make_async_copy(val_hbm.at[pl.ds(rb, NCH)], val_st.at[p], semt).wait()

    def gather_fire(p, kk, b):
        pltpu.async_copy(x_hbm.at[src_st.at[p, kk]], rows_v.at[b], semg.at[b])

    def gather_wait(p, kk, b):
        pltpu.make_async_copy(x_hbm.at[src_st.at[p, kk]], rows_v.at[b],
                              semg.at[b]).wait()

    def scatter_fire(p, kk, b):
        pltpu.async_copy(rows_v.at[b], acc.at[dst_st.at[p, kk]], sems.at[b],
                         add=True)

    def scatter_wait(p, kk, b):
        pltpu.make_async_copy(rows_v.at[b], acc.at[dst_st.at[p, kk]],
                              sems.at[b]).wait()

    lo = c * HALF

    def loc_chunk(p, kk):
        for g in range(CH // 16):
            sl = pl.ds(g * 16, 16)
            d = dst_st[p, kk, sl]
            keep = (d >= lo) & (d < lo + HALF)
            dst_st[p, kk, sl] = jnp.where(d >= HALF, d - HALF, d)
            val_st[p, kk, sl] = jnp.where(
                keep, val_st[p, kk, sl], jnp.zeros((16,), jnp.float32))

    stage_fire(0, 0)
    stage_wait(0)
    for k0 in range(LOOK):
        gather_fire(0, k0, k0)

    def chunk_body(k, carry):
        q = k // NCH
        kk = k - q * NCH
        p = lax.rem(q, 2)
        b = lax.rem(k, NB)

        @pl.when(jnp.logical_and(kk == 0, q + 1 < NST))
        def _fire_stage():
            stage_fire(q + 1, 1 - p)

        @pl.when(jnp.logical_and(kk == NCH - LOOK - 1, q + 1 < NST))
        def _wait_stage():
            stage_wait(1 - p)

        gather_wait(p, kk, b)
        loc_chunk(p, kk)

        for g in range(CH // 16):
            vvals = val_st[p, kk, pl.ds(g * 16, 16)]
            base = g * 16
            for i in range(16):
                vv = jnp.broadcast_to(vvals[i], (16,))
                for j in range(D // 16):
                    sl = pl.ds(j * 16, 16)
                    rows_v[b, base + i, sl] = rows_v[b, base + i, sl] * vv

        scatter_fire(p, kk, b)

        kf = k + LOOK
        @pl.when(kf < NCHT)
        def _fire_next():
            qf = kf // NCH
            kkf = kf - qf * NCH
            pf = lax.rem(qf, 2)
            bf = lax.rem(kf, NB)

            @pl.when(k >= NB - LOOK)
            def _drain_scatter():
                ko = k - (NB - LOOK)
                qo = ko // NCH
                kko = ko - qo * NCH
                po = lax.rem(qo, 2)
                scatter_wait(po, kko, bf)

            gather_fire(pf, kkf, bf)
        return carry

    lax.fori_loop(0, NCHT, chunk_body, 0)

    for kt in range(NCHT - NB, NCHT):
        q = kt // NCH
        kk = kt - q * NCH
        scatter_wait(q % 2, kk, kt % NB)

    plsc.subcore_barrier()

    @pl.when(s < 15)
    def _copy_full():
        pltpu.sync_copy(acc.at[pl.ds(s * ZR, ZR)],
                        out_hbm.at[pl.ds(lo + s * ZR, ZR)])

    @pl.when(s == 15)
    def _copy_tail():
        pltpu.sync_copy(acc.at[pl.ds(15 * ZR, TAIL)],
                        out_hbm.at[pl.ds(lo + 15 * ZR, TAIL)])


_layer = functools.partial(
    pl.kernel,
    out_type=jax.ShapeDtypeStruct((NN, D), jnp.float32),
    mesh=plsc.VectorSubcoreMesh(core_axis_name="c", subcore_axis_name="s",
                                num_cores=2, num_subcores=16),
    compiler_params=pltpu.CompilerParams(use_tc_tiling_on_sc=False),
    scratch_types=[
        pltpu.VMEM((2, NCH, CH), jnp.int32),
        pltpu.VMEM((2, NCH, CH), jnp.int32),
        pltpu.VMEM((2, NCH, CH), jnp.float32),
        pltpu.VMEM((NB, CH, D), jnp.float32),
        pltpu.VMEM_SHARED((ACC_ROWS, D), jnp.float32),
        pltpu.SemaphoreType.DMA((NB,)),
        pltpu.SemaphoreType.DMA((NB,)),
        pltpu.SemaphoreType.DMA,
    ],
)(_layer_body)


def _mean_body(a_ref, b_ref, c_ref, d_ref, o_ref):
    o_ref[...] = (a_ref[...] + b_ref[...] + c_ref[...] + d_ref[...]) * 0.25


def _mean4(a, b, c, d):
    blk = (400, D)
    spec = pl.BlockSpec(blk, lambda i: (i, 0))
    return pl.pallas_call(
        _mean_body,
        grid=(NN // blk[0],),
        in_specs=[spec] * 4,
        out_specs=spec,
        out_shape=jax.ShapeDtypeStruct((NN, D), jnp.float32),
    )(a, b, c, d)


def kernel(adj_indices, adj_values, user_emb, item_emb):
    x0 = jnp.concatenate([user_emb, item_emb], axis=0)
    dst = adj_indices[0].astype(jnp.int32)
    src = adj_indices[1].astype(jnp.int32)
    pad = EP - E
    src2 = jnp.concatenate([src, jnp.zeros((pad,), jnp.int32)]).reshape(EROWS, CH)
    dst2 = jnp.concatenate([dst, jnp.full((pad,), NN, jnp.int32)]).reshape(EROWS, CH)
    val2 = jnp.concatenate([adj_values.astype(jnp.float32),
                            jnp.zeros((pad,), jnp.float32)]).reshape(EROWS, CH)

    x1 = _layer(x0, src2, dst2, val2)
    x2 = _layer(x1, src2, dst2, val2)
    x3 = _layer(x2, src2, dst2, val2)
    out = _mean4(x0, x1, x2, x3)
    return out[:NU], out[NU:]


# speedup vs baseline: 7.5548x; 2.0799x over previous
import functools

import jax
import jax.numpy as jnp
from jax import lax
from jax.experimental import pallas as pl
from jax.experimental.pallas import tpu as pltpu
from jax.experimental.pallas import tpu_sc as plsc

NU = 25000
NI = 25000
NN = NU + NI
D = 64
W = 32
E = 800000

CH = 128
NCH = 4
NST = 100
NCHT = NCH * NST
PT = CH * NCHT
EP = PT * 16
EROWS = EP // CH

NB = 6
LOOK = 4
NSTG = 4

ACC_ROWS = 50048
ZR = ACC_ROWS // 16
TAIL = NN - 15 * ZR


def _layer_body(x_hbm, src_hbm, dst_hbm, val_hbm, out_hbm,
                src_st, dst_st, val_st, rows_v, acc, semg, sems, semt):
    c = lax.axis_index("c")
    s = lax.axis_index("s")
    cbase = c * NN

    def zrow_body(r, carry):
        for j in range(W // 16):
            rows_v[0, r, pl.ds(j * 16, 16)] = jnp.zeros((16,), jnp.float32)
        return carry
    lax.fori_loop(0, CH, zrow_body, 0)
    zbase = s * ZR
    for k in range(ZR // CH):
        pltpu.sync_copy(rows_v.at[0], acc.at[pl.ds(zbase + k * CH, CH)])
    pltpu.sync_copy(rows_v.at[0, pl.ds(0, ZR % CH)],
                    acc.at[pl.ds(zbase + (ZR // CH) * CH, ZR % CH)])
    plsc.subcore_barrier()

    rbase = s * (PT // CH)

    def stage_fire(q):
        ps = lax.rem(q, NSTG)
        rb = rbase + q * NCH
        pltpu.async_copy(src_hbm.at[pl.ds(rb, NCH)], src_st.at[ps], semt)
        pltpu.async_copy(dst_hbm.at[pl.ds(rb, NCH)], dst_st.at[ps], semt)
        pltpu.async_copy(val_hbm.at[pl.ds(rb, NCH)], val_st.at[ps], semt)

    def stage_wait_adjust(q):
        ps = lax.rem(q, NSTG)
        rb = rbase
        pltpu.make_async_copy(src_hbm.at[pl.ds(rb, NCH)], src_st.at[ps], semt).wait()
        pltpu.make_async_copy(dst_hbm.at[pl.ds(rb, NCH)], dst_st.at[ps], semt).wait()
        pltpu.make_async_copy(val_hbm.at[pl.ds(rb, NCH)], val_st.at[ps], semt).wait()
        for kk in range(NCH):
            for g in range(CH // 16):
                sl = pl.ds(g * 16, 16)
                src_st[ps, kk, sl] = src_st[ps, kk, sl] + cbase

    def gather_fire(ps, kk, b):
        pltpu.async_copy(x_hbm.at[src_st.at[ps, kk]], rows_v.at[b], semg.at[b])

    def gather_wait(ps, kk, b):
        pltpu.make_async_copy(x_hbm.at[src_st.at[ps, kk]], rows_v.at[b],
                              semg.at[b]).wait()

    def scatter_fire(ps, kk, b):
        pltpu.async_copy(rows_v.at[b], acc.at[dst_st.at[ps, kk]], sems.at[b],
                         add=True)

    def scatter_wait(ps, kk, b):
        pltpu.make_async_copy(rows_v.at[b], acc.at[dst_st.at[ps, kk]],
                              sems.at[b]).wait()

    stage_fire(0)
    stage_fire(1)
    stage_wait_adjust(0)
    stage_wait_adjust(1)
    stage_fire(2)
    for k0 in range(LOOK):
        gather_fire(k0 // NCH, k0 % NCH, k0)

    def chunk_body(k, carry):
        q = k // NCH
        kk = k - q * NCH
        ps = lax.rem(q, NSTG)
        b = lax.rem(k, NB)

        @pl.when(jnp.logical_and(jnp.logical_and(kk == 0, k > 0),
                                 q + 1 < NST))
        def _stage_ring():
            stage_wait_adjust(q + 1)

            @pl.when(q + 2 < NST)
            def _fire_stage():
                stage_fire(q + 2)

        gather_wait(ps, kk, b)

        for g in range(CH // 16):
            vvals = val_st[ps, kk, pl.ds(g * 16, 16)]
            base = g * 16
            for i in range(16):
                vv = jnp.broadcast_to(vvals[i], (16,))
                for j in range(W // 16):
                    sl = pl.ds(j * 16, 16)
                    rows_v[b, base + i, sl] = rows_v[b, base + i, sl] * vv

        scatter_fire(ps, kk, b)

        kf = k + LOOK
        @pl.when(kf < NCHT)
        def _fire_next():
            qf = kf // NCH
            kkf = kf - qf * NCH
            psf = lax.rem(qf, NSTG)
            bf = lax.rem(kf, NB)

            @pl.when(k >= NB - LOOK)
            def _drain_scatter():
                ko = k - (NB - LOOK)
                qo = ko // NCH
                kko = ko - qo * NCH
                pso = lax.rem(qo, NSTG)
                scatter_wait(pso, kko, bf)

            gather_fire(psf, kkf, bf)
        return carry

    lax.fori_loop(0, NCHT, chunk_body, 0)

    for kt in range(NCHT - NB, NCHT):
        q = kt // NCH
        kk = kt - q * NCH
        scatter_wait(q % NSTG, kk, kt % NB)

    plsc.subcore_barrier()

    @pl.when(s < 15)
    def _copy_full():
        pltpu.sync_copy(acc.at[pl.ds(s * ZR, ZR)],
                        out_hbm.at[pl.ds(cbase + s * ZR, ZR)])

    @pl.when(s == 15)
    def _copy_tail():
        pltpu.sync_copy(acc.at[pl.ds(15 * ZR, TAIL)],
                        out_hbm.at[pl.ds(cbase + 15 * ZR, TAIL)])


_layer = functools.partial(
    pl.kernel,
    out_type=jax.ShapeDtypeStruct((2 * NN, W), jnp.float32),
    mesh=plsc.VectorSubcoreMesh(core_axis_name="c", subcore_axis_name="s",
                                num_cores=2, num_subcores=16),
    compiler_params=pltpu.CompilerParams(use_tc_tiling_on_sc=False),
    scratch_types=[
        pltpu.VMEM((NSTG, NCH, CH), jnp.int32),
        pltpu.VMEM((NSTG, NCH, CH), jnp.int32),
        pltpu.VMEM((NSTG, NCH, CH), jnp.float32),
        pltpu.VMEM((NB, CH, W), jnp.float32),
        pltpu.VMEM_SHARED((ACC_ROWS, W), jnp.float32),
        pltpu.SemaphoreType.DMA((NB,)),
        pltpu.SemaphoreType.DMA((NB,)),
        pltpu.SemaphoreType.DMA,
    ],
)(_layer_body)


def _mean_body(a_ref, b_ref, c_ref, d_ref, o_ref):
    o_ref[...] = (a_ref[...] + b_ref[...] + c_ref[...] + d_ref[...]) * 0.25


def _mean4(a, b, c, d):
    blk = (400, W)
    spec = pl.BlockSpec(blk, lambda i: (i, 0))
    return pl.pallas_call(
        _mean_body,
        grid=(2 * NN // blk[0],),
        in_specs=[spec] * 4,
        out_specs=spec,
        out_shape=jax.ShapeDtypeStruct((2 * NN, W), jnp.float32),
    )(a, b, c, d)


def kernel(adj_indices, adj_values, user_emb, item_emb):
    x0 = jnp.concatenate([user_emb, item_emb], axis=0)
    x0s = jnp.concatenate([x0[:, :W], x0[:, W:]], axis=0)
    dst = adj_indices[0].astype(jnp.int32)
    src = adj_indices[1].astype(jnp.int32)
    pad = EP - E
    src2 = jnp.concatenate([src, jnp.zeros((pad,), jnp.int32)]).reshape(EROWS, CH)
    dst2 = jnp.concatenate([dst, jnp.full((pad,), NN, jnp.int32)]).reshape(EROWS, CH)
    val2 = jnp.concatenate([adj_values.astype(jnp.float32),
                            jnp.zeros((pad,), jnp.float32)]).reshape(EROWS, CH)

    x1 = _layer(x0s, src2, dst2, val2)
    x2 = _layer(x1, src2, dst2, val2)
    x3 = _layer(x2, src2, dst2, val2)
    ms = _mean4(x0s, x1, x2, x3)
    out = jnp.concatenate([ms[:NN], ms[NN:]], axis=1)
    return out[:NU], out[NU:]
